# Initial kernel scaffold; baseline (speedup 1.0000x reference)
import jax
import jax.numpy as jnp
from jax.experimental import pallas as pl


def kernel(x, edge_index, batch, W_red, b_red, ggc_W, gru_Wih, gru_Whh, gru_bih, gru_bhh, gate_W, gate_b, nn_W, nn_b):
    raise NotImplementedError("write your pallas kernel here")


# speedup vs baseline: 1.0878x; 1.0878x over previous
import functools

import jax
import jax.numpy as jnp
from jax import lax
from jax.experimental import pallas as pl
from jax.experimental.pallas import tpu as pltpu

N = 10000
E = 320000
D = 128
G = 16
T = 4
NUM_CLS = 2


def _mm_bias_body(x_ref, w_ref, b_ref, o_ref):
    o_ref[...] = (
        jnp.dot(x_ref[...], w_ref[...], preferred_element_type=jnp.float32)
        + b_ref[...]
    )


def _mm_bias(x, W, b2):
    n, k = x.shape
    m = W.shape[1]
    bn = 2000
    return pl.pallas_call(
        _mm_bias_body,
        grid=(n // bn,),
        in_specs=[
            pl.BlockSpec((bn, k), lambda i: (i, 0)),
            pl.BlockSpec((k, m), lambda i: (0, 0)),
            pl.BlockSpec((1, m), lambda i: (0, 0)),
        ],
        out_specs=pl.BlockSpec((bn, m), lambda i: (i, 0)),
        out_shape=jax.ShapeDtypeStruct((n, m), jnp.float32),
    )(x, W, b2)


def _gru_body(p0_ref, p1_ref, h_ref, wg_ref, wih_ref, whh_ref, bih_ref,
              bhh_ref, o_ref):
    aggh = p0_ref[...] + p1_ref[...]
    msg = jnp.dot(aggh, wg_ref[...], preferred_element_type=jnp.float32)
    gi = jnp.dot(msg, wih_ref[...], preferred_element_type=jnp.float32) + bih_ref[...]
    gh = jnp.dot(h_ref[...], whh_ref[...], preferred_element_type=jnp.float32) + bhh_ref[...]
    i_r, i_z, i_n = gi[:, :D], gi[:, D:2 * D], gi[:, 2 * D:]
    h_r, h_z, h_n = gh[:, :D], gh[:, D:2 * D], gh[:, 2 * D:]
    r = jax.nn.sigmoid(i_r + h_r)
    z = jax.nn.sigmoid(i_z + h_z)
    nn_ = jnp.tanh(i_n + r * h_n)
    h = h_ref[...]
    o_ref[...] = (1.0 - z) * nn_ + z * h


def _gru_step(p0, p1, h, Wg, WihT, WhhT, bih2, bhh2):
    bn = 2000
    return pl.pallas_call(
        _gru_body,
        grid=(N // bn,),
        in_specs=[
            pl.BlockSpec((bn, D), lambda i: (i, 0)),
            pl.BlockSpec((bn, D), lambda i: (i, 0)),
            pl.BlockSpec((bn, D), lambda i: (i, 0)),
            pl.BlockSpec((D, D), lambda i: (0, 0)),
            pl.BlockSpec((D, 3 * D), lambda i: (0, 0)),
            pl.BlockSpec((D, 3 * D), lambda i: (0, 0)),
            pl.BlockSpec((1, 3 * D), lambda i: (0, 0)),
            pl.BlockSpec((1, 3 * D), lambda i: (0, 0)),
        ],
        out_specs=pl.BlockSpec((bn, D), lambda i: (i, 0)),
        out_shape=jax.ShapeDtypeStruct((N, D), jnp.float32),
    )(p0, p1, h, Wg, WihT, WhhT, bih2, bhh2)


def _pool_body(h_ref, b_ref, gw_ref, gb_ref, nw_ref, nb_ref, o_ref):
    h = h_ref[...]
    batch = b_ref[...]
    gidx = lax.broadcasted_iota(jnp.int32, (N, G), 1)
    mask = batch == gidx
    maskf = mask.astype(jnp.float32)
    gate = jnp.dot(h, gw_ref[...], preferred_element_type=jnp.float32) + gb_ref[...]
    gmax = jnp.max(jnp.where(mask, gate, -1e30), axis=0, keepdims=True)
    gsel = jnp.sum(maskf * gmax, axis=1, keepdims=True)
    e = jnp.exp(gate - gsel)
    denom = jnp.sum(maskf * e, axis=0, keepdims=True)
    dsel = jnp.sum(maskf * denom, axis=1, keepdims=True)
    alpha = e / dsel
    feat = jnp.dot(h, nw_ref[...], preferred_element_type=jnp.float32) + nb_ref[...]
    w = alpha * feat
    pooled = lax.dot_general(maskf, w, (((0,), (0,)), ((), ())),
                             preferred_element_type=jnp.float32)
    pm = jnp.max(pooled, axis=1, keepdims=True)
    pe = jnp.exp(pooled - pm)
    o_ref[...] = pe / jnp.sum(pe, axis=1, keepdims=True)


def _pool(h, batch2, gate_W, gate_b2, nn_W, nn_b2):
    return pl.pallas_call(
        _pool_body,
        out_shape=jax.ShapeDtypeStruct((G, NUM_CLS), jnp.float32),
    )(h, batch2, gate_W, gate_b2, nn_W, nn_b2)


def _aggregate(h, src, dst):
    return jax.ops.segment_sum(jnp.take(h, src, axis=0), dst, num_segments=N)


def kernel(x, edge_index, batch, W_red, b_red, ggc_W, gru_Wih, gru_Whh,
           gru_bih, gru_bhh, gate_W, gate_b, nn_W, nn_b):
    src = edge_index[0]
    dst = edge_index[1]
    WihT = gru_Wih.T
    WhhT = gru_Whh.T
    bih2 = gru_bih.reshape(1, 3 * D)
    bhh2 = gru_bhh.reshape(1, 3 * D)
    b_red2 = b_red.reshape(1, D)
    gate_b2 = gate_b.reshape(1, 1)
    nn_b2 = nn_b.reshape(1, NUM_CLS)
    batch2 = batch.reshape(N, 1)

    h = _mm_bias(x, W_red, b_red2)
    for i in range(T):
        agg = _aggregate(h, src, dst)
        zero = jnp.zeros_like(agg)
        h = _gru_step(agg, zero, h, ggc_W[i], WihT, WhhT, bih2, bhh2)
    return _pool(h, batch2, gate_W, gate_b2, nn_W, nn_b2)


# speedup vs baseline: 4.9701x; 4.5689x over previous
import functools

import jax
import jax.numpy as jnp
from jax import lax
from jax.experimental import pallas as pl
from jax.experimental.pallas import tpu as pltpu
from jax.experimental.pallas import tpu_sc as plsc

N = 10000
E = 320000
D = 128
G = 16
T = 4
NUM_CLS = 2


def _mm_bias_body(x_ref, w_ref, b_ref, o_ref):
    o_ref[...] = (
        jnp.dot(x_ref[...], w_ref[...], preferred_element_type=jnp.float32)
        + b_ref[...]
    )


def _mm_bias(x, W, b2):
    n, k = x.shape
    m = W.shape[1]
    bn = 2000
    return pl.pallas_call(
        _mm_bias_body,
        grid=(n // bn,),
        in_specs=[
            pl.BlockSpec((bn, k), lambda i: (i, 0)),
            pl.BlockSpec((k, m), lambda i: (0, 0)),
            pl.BlockSpec((1, m), lambda i: (0, 0)),
        ],
        out_specs=pl.BlockSpec((bn, m), lambda i: (i, 0)),
        out_shape=jax.ShapeDtypeStruct((n, m), jnp.float32),
    )(x, W, b2)


def _gru_body(p0_ref, p1_ref, h_ref, wg_ref, wih_ref, whh_ref, bih_ref,
              bhh_ref, o_ref):
    aggh = p0_ref[...] + p1_ref[...]
    msg = jnp.dot(aggh, wg_ref[...], preferred_element_type=jnp.float32)
    gi = jnp.dot(msg, wih_ref[...], preferred_element_type=jnp.float32) + bih_ref[...]
    gh = jnp.dot(h_ref[...], whh_ref[...], preferred_element_type=jnp.float32) + bhh_ref[...]
    i_r, i_z, i_n = gi[:, :D], gi[:, D:2 * D], gi[:, 2 * D:]
    h_r, h_z, h_n = gh[:, :D], gh[:, D:2 * D], gh[:, 2 * D:]
    r = jax.nn.sigmoid(i_r + h_r)
    z = jax.nn.sigmoid(i_z + h_z)
    nn_ = jnp.tanh(i_n + r * h_n)
    h = h_ref[...]
    o_ref[...] = (1.0 - z) * nn_ + z * h


def _gru_step(p0, p1, h, Wg, WihT, WhhT, bih2, bhh2):
    bn = 2000
    return pl.pallas_call(
        _gru_body,
        grid=(N // bn,),
        in_specs=[
            pl.BlockSpec((bn, D), lambda i: (i, 0)),
            pl.BlockSpec((bn, D), lambda i: (i, 0)),
            pl.BlockSpec((bn, D), lambda i: (i, 0)),
            pl.BlockSpec((D, D), lambda i: (0, 0)),
            pl.BlockSpec((D, 3 * D), lambda i: (0, 0)),
            pl.BlockSpec((D, 3 * D), lambda i: (0, 0)),
            pl.BlockSpec((1, 3 * D), lambda i: (0, 0)),
            pl.BlockSpec((1, 3 * D), lambda i: (0, 0)),
        ],
        out_specs=pl.BlockSpec((bn, D), lambda i: (i, 0)),
        out_shape=jax.ShapeDtypeStruct((N, D), jnp.float32),
    )(p0, p1, h, Wg, WihT, WhhT, bih2, bhh2)


def _pool_body(h_ref, b_ref, gw_ref, gb_ref, nw_ref, nb_ref, o_ref):
    h = h_ref[...]
    batch = b_ref[...]
    gidx = lax.broadcasted_iota(jnp.int32, (N, G), 1)
    mask = batch == gidx
    maskf = mask.astype(jnp.float32)
    gate = jnp.dot(h, gw_ref[...], preferred_element_type=jnp.float32) + gb_ref[...]
    gmax = jnp.max(jnp.where(mask, gate, -1e30), axis=0, keepdims=True)
    gsel = jnp.sum(maskf * gmax, axis=1, keepdims=True)
    e = jnp.exp(gate - gsel)
    denom = jnp.sum(maskf * e, axis=0, keepdims=True)
    dsel = jnp.sum(maskf * denom, axis=1, keepdims=True)
    alpha = e / dsel
    feat = jnp.dot(h, nw_ref[...], preferred_element_type=jnp.float32) + nb_ref[...]
    w = alpha * feat
    pooled = lax.dot_general(maskf, w, (((0,), (0,)), ((), ())),
                             preferred_element_type=jnp.float32)
    pm = jnp.max(pooled, axis=1, keepdims=True)
    pe = jnp.exp(pooled - pm)
    o_ref[...] = pe / jnp.sum(pe, axis=1, keepdims=True)


def _pool(h, batch2, gate_W, gate_b2, nn_W, nn_b2):
    return pl.pallas_call(
        _pool_body,
        out_shape=jax.ShapeDtypeStruct((G, NUM_CLS), jnp.float32),
    )(h, batch2, gate_W, gate_b2, nn_W, nn_b2)


NC = 2
NS = 16
NW = NC * NS
E_PER_TILE = E // NW
CHUNK = 80
NCHUNK = E_PER_TILE // CHUNK
N_PAD = 10240
ROWS_PER_TILE = N_PAD // NS


def _sc_agg_body(h_hbm, src_hbm, dst_hbm, zeros_hbm, out_hbm,
                 shared, src_v, dst_v, rows_v, sem):
    cid = lax.axis_index("c")
    sid = lax.axis_index("s")
    wid = sid * NC + cid
    base = wid * E_PER_TILE
    r0 = sid * ROWS_PER_TILE
    pltpu.sync_copy(zeros_hbm.at[pl.ds(r0, ROWS_PER_TILE)],
                    shared.at[pl.ds(r0, ROWS_PER_TILE)])
    plsc.subcore_barrier()

    def chunk_body(j, carry):
        b = base + j * CHUNK
        pltpu.sync_copy(src_hbm.at[pl.ds(b, CHUNK)], src_v)
        pltpu.sync_copy(dst_hbm.at[pl.ds(b, CHUNK)], dst_v)
        pltpu.async_copy(h_hbm.at[src_v], rows_v, sem).wait()
        pltpu.sync_copy(rows_v, shared.at[dst_v], add=True)
        return carry

    lax.fori_loop(0, NCHUNK, chunk_body, 0)
    plsc.subcore_barrier()
    pltpu.sync_copy(shared.at[pl.ds(r0, ROWS_PER_TILE)],
                    out_hbm.at[pl.ds(cid * N_PAD + r0, ROWS_PER_TILE)])


def _sc_aggregate(h, src, dst, zeros):
    mesh = plsc.VectorSubcoreMesh(core_axis_name="c", subcore_axis_name="s")
    k = pl.kernel(
        _sc_agg_body,
        mesh=mesh,
        out_type=jax.ShapeDtypeStruct((NC * N_PAD, D), jnp.float32),
        scratch_types=[
            pltpu.VMEM_SHARED((N_PAD, D), jnp.float32),
            pltpu.VMEM((CHUNK,), jnp.int32),
            pltpu.VMEM((CHUNK,), jnp.int32),
            pltpu.VMEM((CHUNK, D), jnp.float32),
            pltpu.SemaphoreType.DMA,
        ],
    )
    return k(h, src, dst, zeros)


def kernel(x, edge_index, batch, W_red, b_red, ggc_W, gru_Wih, gru_Whh,
           gru_bih, gru_bhh, gate_W, gate_b, nn_W, nn_b):
    src = edge_index[0]
    dst = edge_index[1]
    WihT = gru_Wih.T
    WhhT = gru_Whh.T
    bih2 = gru_bih.reshape(1, 3 * D)
    bhh2 = gru_bhh.reshape(1, 3 * D)
    b_red2 = b_red.reshape(1, D)
    gate_b2 = gate_b.reshape(1, 1)
    nn_b2 = nn_b.reshape(1, NUM_CLS)
    batch2 = batch.reshape(N, 1)

    h = _mm_bias(x, W_red, b_red2)
    zeros = jnp.zeros((N_PAD, D), jnp.float32)
    for i in range(T):
        parts = _sc_aggregate(h, src, dst, zeros)
        h = _gru_step(parts[:N], parts[N_PAD:N_PAD + N], h, ggc_W[i],
                      WihT, WhhT, bih2, bhh2)
    return _pool(h, batch2, gate_W, gate_b2, nn_W, nn_b2)


# speedup vs baseline: 9.2047x; 1.8520x over previous
import functools

import jax
import jax.numpy as jnp
from jax import lax
from jax.experimental import pallas as pl
from jax.experimental.pallas import tpu as pltpu
from jax.experimental.pallas import tpu_sc as plsc

N = 10000
E = 320000
D = 128
G = 16
T = 4
NUM_CLS = 2


def _mm_bias_body(x_ref, w_ref, b_ref, o_ref):
    o_ref[...] = (
        jnp.dot(x_ref[...], w_ref[...], preferred_element_type=jnp.float32)
        + b_ref[...]
    )


def _mm_bias(x, W, b2):
    n, k = x.shape
    m = W.shape[1]
    bn = 2000
    return pl.pallas_call(
        _mm_bias_body,
        grid=(n // bn,),
        in_specs=[
            pl.BlockSpec((bn, k), lambda i: (i, 0)),
            pl.BlockSpec((k, m), lambda i: (0, 0)),
            pl.BlockSpec((1, m), lambda i: (0, 0)),
        ],
        out_specs=pl.BlockSpec((bn, m), lambda i: (i, 0)),
        out_shape=jax.ShapeDtypeStruct((n, m), jnp.float32),
    )(x, W, b2)


def _gru_body(p0_ref, p1_ref, h_ref, wg_ref, wih_ref, whh_ref, bih_ref,
              bhh_ref, o_ref):
    aggh = p0_ref[...] + p1_ref[...]
    msg = jnp.dot(aggh, wg_ref[...], preferred_element_type=jnp.float32)
    gi = jnp.dot(msg, wih_ref[...], preferred_element_type=jnp.float32) + bih_ref[...]
    gh = jnp.dot(h_ref[...], whh_ref[...], preferred_element_type=jnp.float32) + bhh_ref[...]
    i_r, i_z, i_n = gi[:, :D], gi[:, D:2 * D], gi[:, 2 * D:]
    h_r, h_z, h_n = gh[:, :D], gh[:, D:2 * D], gh[:, 2 * D:]
    r = jax.nn.sigmoid(i_r + h_r)
    z = jax.nn.sigmoid(i_z + h_z)
    nn_ = jnp.tanh(i_n + r * h_n)
    h = h_ref[...]
    o_ref[...] = (1.0 - z) * nn_ + z * h


def _gru_step(p0, p1, h, Wg, WihT, WhhT, bih2, bhh2):
    bn = 2000
    return pl.pallas_call(
        _gru_body,
        grid=(N // bn,),
        in_specs=[
            pl.BlockSpec((bn, D), lambda i: (i, 0)),
            pl.BlockSpec((bn, D), lambda i: (i, 0)),
            pl.BlockSpec((bn, D), lambda i: (i, 0)),
            pl.BlockSpec((D, D), lambda i: (0, 0)),
            pl.BlockSpec((D, 3 * D), lambda i: (0, 0)),
            pl.BlockSpec((D, 3 * D), lambda i: (0, 0)),
            pl.BlockSpec((1, 3 * D), lambda i: (0, 0)),
            pl.BlockSpec((1, 3 * D), lambda i: (0, 0)),
        ],
        out_specs=pl.BlockSpec((bn, D), lambda i: (i, 0)),
        out_shape=jax.ShapeDtypeStruct((N, D), jnp.float32),
    )(p0, p1, h, Wg, WihT, WhhT, bih2, bhh2)


def _pool_body(h_ref, b_ref, gw_ref, gb_ref, nw_ref, nb_ref, o_ref):
    h = h_ref[...]
    batch = b_ref[...]
    gidx = lax.broadcasted_iota(jnp.int32, (N, G), 1)
    mask = batch == gidx
    maskf = mask.astype(jnp.float32)
    gate = jnp.dot(h, gw_ref[...], preferred_element_type=jnp.float32) + gb_ref[...]
    gmax = jnp.max(jnp.where(mask, gate, -1e30), axis=0, keepdims=True)
    gsel = jnp.sum(maskf * gmax, axis=1, keepdims=True)
    e = jnp.exp(gate - gsel)
    denom = jnp.sum(maskf * e, axis=0, keepdims=True)
    dsel = jnp.sum(maskf * denom, axis=1, keepdims=True)
    alpha = e / dsel
    feat = jnp.dot(h, nw_ref[...], preferred_element_type=jnp.float32) + nb_ref[...]
    w = alpha * feat
    pooled = lax.dot_general(maskf, w, (((0,), (0,)), ((), ())),
                             preferred_element_type=jnp.float32)
    pm = jnp.max(pooled, axis=1, keepdims=True)
    pe = jnp.exp(pooled - pm)
    o_ref[...] = pe / jnp.sum(pe, axis=1, keepdims=True)


def _pool(h, batch2, gate_W, gate_b2, nn_W, nn_b2):
    return pl.pallas_call(
        _pool_body,
        out_shape=jax.ShapeDtypeStruct((G, NUM_CLS), jnp.float32),
    )(h, batch2, gate_W, gate_b2, nn_W, nn_b2)


NC = 2
NS = 16
NW = NC * NS
E_PER_TILE = E // NW
CHUNK = 128
NPC = 80
PAD_E = NPC * CHUNK - E_PER_TILE
N_PAD = 10240
ROWS_PER_TILE = N_PAD // NS
HALF_ROWS = NPC


def _sc_agg_body(h_hbm, ei_hbm, zeros_hbm, out_hbm,
                 shared, ei_h, rows0, rows1,
                 gs0, gs1, ss0, ss1):
    cid = lax.axis_index("c")
    sid = lax.axis_index("s")
    wid = sid * NC + cid
    r0 = sid * ROWS_PER_TILE
    pltpu.sync_copy(zeros_hbm.at[pl.ds(r0, ROWS_PER_TILE)],
                    shared.at[pl.ds(r0, ROWS_PER_TILE)])
    plsc.subcore_barrier()

    for half in range(2):
        pltpu.sync_copy(ei_hbm.at[wid].at[pl.ds(half * HALF_ROWS, HALF_ROWS)],
                        ei_h)

        def pair_body(p, carry):
            q = 4 * p
            g0 = pltpu.async_copy(h_hbm.at[ei_h.at[q]], rows0, gs0)
            g1 = pltpu.async_copy(h_hbm.at[ei_h.at[q + 2]], rows1, gs1)
            g0.wait()
            s0 = pltpu.async_copy(rows0, shared.at[ei_h.at[q + 1]], ss0,
                                  add=True)
            g1.wait()
            s1 = pltpu.async_copy(rows1, shared.at[ei_h.at[q + 3]], ss1,
                                  add=True)
            s0.wait()
            s1.wait()
            return carry

        lax.fori_loop(0, HALF_ROWS // 4, pair_body, 0)
    plsc.subcore_barrier()
    pltpu.sync_copy(shared.at[pl.ds(r0, ROWS_PER_TILE)],
                    out_hbm.at[pl.ds(cid * N_PAD + r0, ROWS_PER_TILE)])


def _sc_aggregate(h, ei, zeros):
    mesh = plsc.VectorSubcoreMesh(core_axis_name="c", subcore_axis_name="s")
    k = pl.kernel(
        _sc_agg_body,
        mesh=mesh,
        out_type=jax.ShapeDtypeStruct((NC * N_PAD, D), jnp.float32),
        scratch_types=[
            pltpu.VMEM_SHARED((N_PAD, D), jnp.float32),
            pltpu.VMEM((HALF_ROWS, CHUNK), jnp.int32),
            pltpu.VMEM((CHUNK, D), jnp.float32),
            pltpu.VMEM((CHUNK, D), jnp.float32),
            pltpu.SemaphoreType.DMA,
            pltpu.SemaphoreType.DMA,
            pltpu.SemaphoreType.DMA,
            pltpu.SemaphoreType.DMA,
        ],
    )
    return k(h, ei, zeros)


def _build_edge_chunks(edge_index):
    src2 = edge_index[0].reshape(NW, E_PER_TILE)
    dst2 = edge_index[1].reshape(NW, E_PER_TILE)
    pad = jnp.arange(NW * PAD_E, dtype=jnp.int32).reshape(NW, PAD_E)
    pad_src = (pad * 131) % N
    pad_dst = N + pad % (N_PAD - N)
    srcp = jnp.concatenate([src2, pad_src], axis=1).reshape(NW, NPC, CHUNK)
    dstp = jnp.concatenate([dst2, pad_dst], axis=1).reshape(NW, NPC, CHUNK)
    return jnp.stack([srcp, dstp], axis=2).reshape(NW, 2 * NPC, CHUNK)


def kernel(x, edge_index, batch, W_red, b_red, ggc_W, gru_Wih, gru_Whh,
           gru_bih, gru_bhh, gate_W, gate_b, nn_W, nn_b):
    ei = _build_edge_chunks(edge_index)
    WihT = gru_Wih.T
    WhhT = gru_Whh.T
    bih2 = gru_bih.reshape(1, 3 * D)
    bhh2 = gru_bhh.reshape(1, 3 * D)
    b_red2 = b_red.reshape(1, D)
    gate_b2 = gate_b.reshape(1, 1)
    nn_b2 = nn_b.reshape(1, NUM_CLS)
    batch2 = batch.reshape(N, 1)

    h = _mm_bias(x, W_red, b_red2)
    zeros = jnp.zeros((N_PAD, D), jnp.float32)
    for i in range(T):
        parts = _sc_aggregate(h, ei, zeros)
        h = _gru_step(parts[:N], parts[N_PAD:N_PAD + N], h, ggc_W[i],
                      WihT, WhhT, bih2, bhh2)
    return _pool(h, batch2, gate_W, gate_b2, nn_W, nn_b2)


# speedup vs baseline: 9.3492x; 1.0157x over previous
import functools

import jax
import jax.numpy as jnp
from jax import lax
from jax.experimental import pallas as pl
from jax.experimental.pallas import tpu as pltpu
from jax.experimental.pallas import tpu_sc as plsc

N = 10000
E = 320000
D = 128
G = 16
T = 4
NUM_CLS = 2


def _mm_bias_body(x_ref, w_ref, b_ref, o_ref):
    o_ref[...] = (
        jnp.dot(x_ref[...], w_ref[...], preferred_element_type=jnp.float32)
        + b_ref[...]
    )


def _mm_bias(x, W, b2):
    n, k = x.shape
    m = W.shape[1]
    bn = 2000
    return pl.pallas_call(
        _mm_bias_body,
        grid=(n // bn,),
        in_specs=[
            pl.BlockSpec((bn, k), lambda i: (i, 0)),
            pl.BlockSpec((k, m), lambda i: (0, 0)),
            pl.BlockSpec((1, m), lambda i: (0, 0)),
        ],
        out_specs=pl.BlockSpec((bn, m), lambda i: (i, 0)),
        out_shape=jax.ShapeDtypeStruct((n, m), jnp.float32),
    )(x, W, b2)


def _gru_body(p0_ref, p1_ref, h_ref, wg_ref, wih_ref, whh_ref, bih_ref,
              bhh_ref, o_ref):
    aggh = p0_ref[...] + p1_ref[...]
    msg = jnp.dot(aggh, wg_ref[...], preferred_element_type=jnp.float32)
    gi = jnp.dot(msg, wih_ref[...], preferred_element_type=jnp.float32) + bih_ref[...]
    gh = jnp.dot(h_ref[...], whh_ref[...], preferred_element_type=jnp.float32) + bhh_ref[...]
    i_r, i_z, i_n = gi[:, :D], gi[:, D:2 * D], gi[:, 2 * D:]
    h_r, h_z, h_n = gh[:, :D], gh[:, D:2 * D], gh[:, 2 * D:]
    r = jax.nn.sigmoid(i_r + h_r)
    z = jax.nn.sigmoid(i_z + h_z)
    nn_ = jnp.tanh(i_n + r * h_n)
    h = h_ref[...]
    o_ref[...] = (1.0 - z) * nn_ + z * h


def _gru_step(p0, p1, h, Wg, WihT, WhhT, bih2, bhh2):
    bn = 2000
    return pl.pallas_call(
        _gru_body,
        grid=(N // bn,),
        in_specs=[
            pl.BlockSpec((bn, D), lambda i: (i, 0)),
            pl.BlockSpec((bn, D), lambda i: (i, 0)),
            pl.BlockSpec((bn, D), lambda i: (i, 0)),
            pl.BlockSpec((D, D), lambda i: (0, 0)),
            pl.BlockSpec((D, 3 * D), lambda i: (0, 0)),
            pl.BlockSpec((D, 3 * D), lambda i: (0, 0)),
            pl.BlockSpec((1, 3 * D), lambda i: (0, 0)),
            pl.BlockSpec((1, 3 * D), lambda i: (0, 0)),
        ],
        out_specs=pl.BlockSpec((bn, D), lambda i: (i, 0)),
        out_shape=jax.ShapeDtypeStruct((N, D), jnp.float32),
    )(p0, p1, h, Wg, WihT, WhhT, bih2, bhh2)


def _pool_body(h_ref, b_ref, gw_ref, gb_ref, nw_ref, nb_ref, o_ref):
    h = h_ref[...]
    batch = b_ref[...]
    gidx = lax.broadcasted_iota(jnp.int32, (N, G), 1)
    mask = batch == gidx
    maskf = mask.astype(jnp.float32)
    gate = jnp.dot(h, gw_ref[...], preferred_element_type=jnp.float32) + gb_ref[...]
    gmax = jnp.max(jnp.where(mask, gate, -1e30), axis=0, keepdims=True)
    gsel = jnp.sum(maskf * gmax, axis=1, keepdims=True)
    e = jnp.exp(gate - gsel)
    denom = jnp.sum(maskf * e, axis=0, keepdims=True)
    dsel = jnp.sum(maskf * denom, axis=1, keepdims=True)
    alpha = e / dsel
    feat = jnp.dot(h, nw_ref[...], preferred_element_type=jnp.float32) + nb_ref[...]
    w = alpha * feat
    pooled = lax.dot_general(maskf, w, (((0,), (0,)), ((), ())),
                             preferred_element_type=jnp.float32)
    pm = jnp.max(pooled, axis=1, keepdims=True)
    pe = jnp.exp(pooled - pm)
    o_ref[...] = pe / jnp.sum(pe, axis=1, keepdims=True)


def _pool(h, batch2, gate_W, gate_b2, nn_W, nn_b2):
    return pl.pallas_call(
        _pool_body,
        out_shape=jax.ShapeDtypeStruct((G, NUM_CLS), jnp.float32),
    )(h, batch2, gate_W, gate_b2, nn_W, nn_b2)


NC = 2
NS = 16
NW = NC * NS
E_PER_TILE = E // NW
CHUNK = 128
NPC = 80
PAD_E = NPC * CHUNK - E_PER_TILE
N_PAD = 10240
ROWS_PER_TILE = N_PAD // NS
HALF_ROWS = NPC


def _sc_agg_body(h_hbm, ei_hbm, zeros_hbm, out_hbm,
                 shared, ei_h, rows0, rows1,
                 gs0, gs1, ss0, ss1):
    cid = lax.axis_index("c")
    sid = lax.axis_index("s")
    wid = sid * NC + cid
    r0 = sid * ROWS_PER_TILE
    pltpu.sync_copy(zeros_hbm.at[pl.ds(r0, ROWS_PER_TILE)],
                    shared.at[pl.ds(r0, ROWS_PER_TILE)])
    plsc.subcore_barrier()

    scatter_bytes = CHUNK * D * 4
    for half in range(2):
        pltpu.sync_copy(ei_hbm.at[wid].at[pl.ds(half * HALF_ROWS, HALF_ROWS)],
                        ei_h)

        def pair_body(p, carry):
            q = 4 * p
            qm = 4 * (p - 1)

            @pl.when(p > 0)
            def _w0():
                pltpu.make_async_copy(rows0, shared.at[ei_h.at[qm + 1]],
                                      ss0).wait()

            g0 = pltpu.async_copy(h_hbm.at[ei_h.at[q]], rows0, gs0)

            @pl.when(p > 0)
            def _w1():
                pltpu.make_async_copy(rows1, shared.at[ei_h.at[qm + 3]],
                                      ss1).wait()

            g1 = pltpu.async_copy(h_hbm.at[ei_h.at[q + 2]], rows1, gs1)
            g0.wait()
            pltpu.async_copy(rows0, shared.at[ei_h.at[q + 1]], ss0, add=True)
            g1.wait()
            pltpu.async_copy(rows1, shared.at[ei_h.at[q + 3]], ss1, add=True)
            return carry

        lax.fori_loop(0, HALF_ROWS // 4, pair_body, 0)
        qlast = HALF_ROWS - 4
        pltpu.make_async_copy(rows0, shared.at[ei_h.at[qlast + 1]], ss0).wait()
        pltpu.make_async_copy(rows1, shared.at[ei_h.at[qlast + 3]], ss1).wait()
    plsc.subcore_barrier()
    pltpu.sync_copy(shared.at[pl.ds(r0, ROWS_PER_TILE)],
                    out_hbm.at[pl.ds(cid * N_PAD + r0, ROWS_PER_TILE)])


def _sc_aggregate(h, ei, zeros):
    mesh = plsc.VectorSubcoreMesh(core_axis_name="c", subcore_axis_name="s")
    k = pl.kernel(
        _sc_agg_body,
        mesh=mesh,
        out_type=jax.ShapeDtypeStruct((NC * N_PAD, D), jnp.float32),
        scratch_types=[
            pltpu.VMEM_SHARED((N_PAD, D), jnp.float32),
            pltpu.VMEM((HALF_ROWS, CHUNK), jnp.int32),
            pltpu.VMEM((CHUNK, D), jnp.float32),
            pltpu.VMEM((CHUNK, D), jnp.float32),
            pltpu.SemaphoreType.DMA,
            pltpu.SemaphoreType.DMA,
            pltpu.SemaphoreType.DMA,
            pltpu.SemaphoreType.DMA,
        ],
    )
    return k(h, ei, zeros)


def _build_edge_chunks(edge_index):
    src2 = edge_index[0].reshape(NW, E_PER_TILE)
    dst2 = edge_index[1].reshape(NW, E_PER_TILE)
    pad = jnp.arange(NW * PAD_E, dtype=jnp.int32).reshape(NW, PAD_E)
    pad_src = (pad * 131) % N
    pad_dst = N + pad % (N_PAD - N)
    srcp = jnp.concatenate([src2, pad_src], axis=1).reshape(NW, NPC, CHUNK)
    dstp = jnp.concatenate([dst2, pad_dst], axis=1).reshape(NW, NPC, CHUNK)
    return jnp.stack([srcp, dstp], axis=2).reshape(NW, 2 * NPC, CHUNK)


def kernel(x, edge_index, batch, W_red, b_red, ggc_W, gru_Wih, gru_Whh,
           gru_bih, gru_bhh, gate_W, gate_b, nn_W, nn_b):
    ei = _build_edge_chunks(edge_index)
    WihT = gru_Wih.T
    WhhT = gru_Whh.T
    bih2 = gru_bih.reshape(1, 3 * D)
    bhh2 = gru_bhh.reshape(1, 3 * D)
    b_red2 = b_red.reshape(1, D)
    gate_b2 = gate_b.reshape(1, 1)
    nn_b2 = nn_b.reshape(1, NUM_CLS)
    batch2 = batch.reshape(N, 1)

    h = _mm_bias(x, W_red, b_red2)
    zeros = jnp.zeros((N_PAD, D), jnp.float32)
    for i in range(T):
        parts = _sc_aggregate(h, ei, zeros)
        h = _gru_step(parts[:N], parts[N_PAD:N_PAD + N], h, ggc_W[i],
                      WihT, WhhT, bih2, bhh2)
    return _pool(h, batch2, gate_W, gate_b2, nn_W, nn_b2)
